# baseline (speedup 1.0000x reference)
import functools

import jax
import jax.numpy as jnp
from jax import lax
from jax.experimental import pallas as pl
from jax.experimental.pallas import tpu as pltpu
from jax.experimental.pallas import tpu_sc as plsc

_EPS2 = 1e-16
_MAGIC = 0x5F3759DF


def _rsqrt(m):
    i = plsc.bitcast(m, jnp.int32)
    y = plsc.bitcast(_MAGIC - (i >> 1), jnp.float32)
    for _ in range(3):
        y = y * (1.5 - 0.5 * m * y * y)
    return y


def _make_sc_call(B):
    info = plsc.get_sparse_core_info()
    NC, NS, L = info.num_cores, info.num_subcores, info.num_lanes
    NW = NC * NS
    chunk = B // NW

    mesh = plsc.VectorSubcoreMesh(core_axis_name="c", subcore_axis_name="s")
    f32, i32 = jnp.float32, jnp.int32

    @functools.partial(
        pl.kernel,
        mesh=mesh,
        out_type=jax.ShapeDtypeStruct((B,), f32),
        scratch_types=[pltpu.VMEM((chunk,), i32) for _ in range(8)]
        + [pltpu.VMEM((8,), f32), pltpu.VMEM((7, 64), f32),
           pltpu.VMEM((8, 64), f32), pltpu.VMEM((17, 512), f32),
           pltpu.VMEM((4,), f32), pltpu.VMEM((4,), f32), pltpu.VMEM((1,), f32),
           pltpu.VMEM((16,), f32), pltpu.VMEM((16, 16), f32),
           pltpu.VMEM((32,), f32), pltpu.VMEM((40, 32), f32),
           pltpu.MemorySpace.VMEM_SHARED((16, 16), f32),
           pltpu.MemorySpace.VMEM_SHARED((40, 32), f32),
           pltpu.VMEM((chunk,), f32), pltpu.SemaphoreType.DMA],
        compiler_params=pltpu.CompilerParams(needs_layout_passes=False),
    )
    def sc(ga, sa, ca, ma, gb, sb, cb, mb, gw, cw, sw, mw, wv, fcw, fcb, out,
           ga_v, sa_v, ca_v, ma_v, gb_v, sb_v, cb_v, mb_v,
           gw_v, cw_v, sw_v, mw_v, wv_v, fcw_v, fcb_v,
           npub_v, nall_v, ppub_v, tbl_v, shared_n, shared_t, out_v, sem):
        s = lax.axis_index("s")
        cx = lax.axis_index("c")
        wid = s * NC + cx
        base = wid * chunk
        sl_h = pl.ds(base, chunk)
        zero = jnp.zeros((L,), f32)
        lane = lax.broadcasted_iota(i32, (L,), 0)
        lane4 = jnp.minimum(lane, 3)
        in4 = (lane < 4).astype(f32)

        copies = [
            pltpu.async_copy(ga.at[sl_h], ga_v, sem),
            pltpu.async_copy(sa.at[sl_h], sa_v, sem),
            pltpu.async_copy(ca.at[sl_h], ca_v, sem),
            pltpu.async_copy(ma.at[sl_h], ma_v, sem),
            pltpu.async_copy(gb.at[sl_h], gb_v, sem),
            pltpu.async_copy(sb.at[sl_h], sb_v, sem),
            pltpu.async_copy(cb.at[sl_h], cb_v, sem),
            pltpu.async_copy(mb.at[sl_h], mb_v, sem),
            pltpu.async_copy(gw, gw_v, sem),
            pltpu.async_copy(cw, cw_v, sem),
            pltpu.async_copy(sw, sw_v, sem),
            pltpu.async_copy(mw, mw_v, sem),
            pltpu.async_copy(wv, wv_v, sem),
            pltpu.async_copy(fcw, fcw_v, sem),
            pltpu.async_copy(fcb, fcb_v, sem),
        ]
        for cp in copies:
            cp.wait()

        def splat_i(x):
            return jnp.full((L,), x, i32)

        def grow(r):
            return plsc.load_gather(gw_v, [splat_i(4 * r) + lane4]) * in4

        def dot_m(ra, rb):
            acc = zero
            for ch in range(32):
                acc = acc + (mw_v[ra, pl.ds(ch * L, L)]
                             * mw_v[rb, pl.ds(ch * L, L)])
            return jnp.sum(acc)

        def dot4(ref, ra, rb):
            acc = zero
            for ch in range(4):
                acc = acc + (ref[ra, pl.ds(ch * L, L)]
                             * ref[rb, pl.ds(ch * L, L)])
            return jnp.sum(acc)

        def rn_from(n2):
            return _rsqrt(jnp.full((L,), jnp.maximum(n2, _EPS2), f32))

        pv = jnp.where(lane == 0, rn_from(dot_m(s, s)), 0.0)
        pv = jnp.where(lane == 1, rn_from(dot_m(16, 16)), pv)
        npub_v[...] = pv

        @pl.when(s < 7)
        def _():
            npub_v[...] = jnp.where(lane == 2, rn_from(dot4(cw_v, s, s)),
                                    npub_v[...])

        @pl.when(s < 8)
        def _():
            npub_v[...] = jnp.where(lane == 3, rn_from(dot4(sw_v, s, s)),
                                    npub_v[...])

        @pl.when(s == 8)
        def _():
            g0 = grow(0)
            g1 = grow(1)
            npub_v[...] = jnp.where(lane == 4, rn_from(jnp.sum(g0 * g0)),
                                    jnp.where(lane == 5,
                                              rn_from(jnp.sum(g1 * g1)),
                                              npub_v[...]))

        pltpu.sync_copy(npub_v, shared_n.at[s])
        plsc.subcore_barrier()
        pltpu.sync_copy(shared_n, nall_v)

        lane0 = (lane == 0).astype(f32)

        def _splat0(v):
            return jnp.full((L,), jnp.sum(v * lane0), f32)

        def rn_at(row, ln):
            return _splat0(plsc.load_gather(nall_v,
                                            [splat_i(row), splat_i(ln)]))

        def scale(k):
            kv = splat_i(k)
            return _splat0(plsc.load_gather(wv_v, [kv])
                           * plsc.load_gather(fcw_v, [kv]))

        a3 = scale(3)
        rn_ms = rn_at(s, 0)
        rn_m16 = rn_at(0, 1)
        arow = [mw_v[s, pl.ds(ch * L, L)] for ch in range(32)]
        lo = zero
        hi = zero
        for j in range(17):
            acc = zero
            for ch in range(32):
                acc = acc + arow[ch] * mw_v[j, pl.ds(ch * L, L)]
            rn_j = rn_at(j, 0) if j < 16 else rn_m16
            val = jnp.full((L,), jnp.sum(acc), f32) * rn_ms * rn_j * a3
            if j < 16:
                lo = jnp.where(lane == j, val, lo)
            else:
                hi = jnp.where(lane == 0, val, hi)
        v16s = jnp.full((L,), dot_m(16, s), f32) * rn_m16 * rn_ms * a3
        hi = jnp.where(lane == 1, v16s, hi)
        v1616 = jnp.full((L,), dot_m(16, 16), f32) * rn_m16 * rn_m16 * a3
        hi = jnp.where(lane == 2, v1616, hi)
        ppub_v[pl.ds(0, L)] = lo
        ppub_v[pl.ds(L, L)] = hi
        pltpu.sync_copy(ppub_v, shared_t.at[s])

        @pl.when(s < 7)
        def _():
            a1 = scale(1)
            rn_cs = rn_at(s, 2)
            clo = zero
            for j in range(7):
                val = (jnp.full((L,), dot4(cw_v, s, j), f32)
                       * rn_cs * rn_at(j, 2) * a1)
                clo = jnp.where(lane == j, val, clo)
            ppub_v[pl.ds(0, L)] = clo
            ppub_v[pl.ds(L, L)] = zero
            pltpu.sync_copy(ppub_v, shared_t.at[16 + s])

        @pl.when(s < 8)
        def _():
            a2 = scale(2)
            rn_ss = rn_at(s, 3)
            slo = zero
            for j in range(8):
                val = (jnp.full((L,), dot4(sw_v, s, j), f32)
                       * rn_ss * rn_at(j, 3) * a2)
                slo = jnp.where(lane == j, val, slo)
            ppub_v[pl.ds(0, L)] = slo
            ppub_v[pl.ds(L, L)] = zero
            pltpu.sync_copy(ppub_v, shared_t.at[24 + s])

        @pl.when(s == 8)
        def _():
            a0 = scale(0)
            biasv = _splat0(plsc.load_gather(fcb_v, [jnp.zeros((L,), i32)]))
            rows = [grow(0), grow(1)]
            glo = zero
            for i in range(2):
                for j in range(2):
                    dt = jnp.sum(rows[i] * rows[j])
                    val = (jnp.full((L,), dt, f32)
                           * rn_at(8, 4 + i) * rn_at(8, 4 + j) * a0 + biasv)
                    glo = jnp.where(lane == i * 2 + j, val, glo)
            ppub_v[pl.ds(0, L)] = glo
            ppub_v[pl.ds(L, L)] = zero
            pltpu.sync_copy(ppub_v, shared_t.at[32])

        plsc.subcore_barrier()
        pltpu.sync_copy(shared_t, tbl_v)

        c32 = splat_i(32)
        for r in range(chunk // L):
            sl = pl.ds(r * L, L)
            ia = ma_v[sl]
            jb = mb_v[sl]
            i_lt = ia < 16
            j_lt = jb < 16
            mrow = jnp.where(i_lt, ia, jnp.where(j_lt, jb, 0))
            mcol = jnp.where(i_lt, jb, jnp.where(j_lt, 17, 18))
            v = (plsc.load_gather(tbl_v, [c32, ga_v[sl] * 2 + gb_v[sl]])
                 + plsc.load_gather(tbl_v, [ca_v[sl] + 16, cb_v[sl]])
                 + plsc.load_gather(tbl_v, [sa_v[sl] + 24, sb_v[sl]])
                 + plsc.load_gather(tbl_v, [mrow, mcol]))
            out_v[sl] = 1.0 / (1.0 + jnp.exp(-v))
        pltpu.sync_copy(out_v, out.at[sl_h])

    return sc


def kernel(gA, sA, cA, mA, gB, sB, cB, mB,
           gender_W, college_W, school_W, mbti_W, weight, fc_W, fc_b):
    B = gA.shape[0]
    i32 = jnp.int32
    out = _make_sc_call(B)(
        gA.astype(i32), sA.astype(i32), cA.astype(i32), mA.astype(i32),
        gB.astype(i32), sB.astype(i32), cB.astype(i32), mB.astype(i32),
        gender_W.reshape(-1), college_W, school_W, mbti_W,
        weight, fc_W.reshape(-1), fc_b)
    return out.reshape(B, 1)


# speedup vs baseline: 1.2890x; 1.2890x over previous
import functools

import jax
import jax.numpy as jnp
from jax import lax
from jax.experimental import pallas as pl
from jax.experimental.pallas import tpu as pltpu
from jax.experimental.pallas import tpu_sc as plsc

_EPS = 1e-8

_OFF_G = 0
_OFF_C = 4
_OFF_S = 53
_OFF_M = 117
_TBL = 416


def _prep_body(gw, cw, sw, mw, w, fcw, fcb, tg, tc_, ts, tm):
    def cosmat(W):
        n = jnp.maximum(jnp.sqrt(jnp.sum(W * W, axis=1, keepdims=True)), _EPS)
        Wn = W / n
        return lax.dot_general(Wn, Wn, (((1,), (1,)), ((), ())),
                               preferred_element_type=jnp.float32)

    a = w[...] * fcw[...]

    tg[...] = cosmat(gw[...]) * a[0:1] + fcb[...]
    tc_[...] = cosmat(cw[...]) * a[1:2]
    ts[...] = cosmat(sw[...]) * a[2:3]
    tm[...] = cosmat(mw[...]) * a[3:4]


def _prep_call(gender_W, college_W, school_W, mbti_W, weight, fc_W, fc_b):
    vmem = pl.BlockSpec(memory_space=pltpu.VMEM)
    return pl.pallas_call(
        _prep_body,
        out_shape=[
            jax.ShapeDtypeStruct((2, 2), jnp.float32),
            jax.ShapeDtypeStruct((7, 7), jnp.float32),
            jax.ShapeDtypeStruct((8, 8), jnp.float32),
            jax.ShapeDtypeStruct((17, 17), jnp.float32),
        ],
        in_specs=[vmem] * 7,
        out_specs=[vmem, vmem, vmem, vmem],
    )(gender_W, college_W, school_W, mbti_W, weight, fc_W.reshape(-1), fc_b)


def _make_sc_call(B):
    info = plsc.get_sparse_core_info()
    NC, NS, L = info.num_cores, info.num_subcores, info.num_lanes
    NW = NC * NS
    chunk = B // NW

    mesh = plsc.VectorSubcoreMesh(core_axis_name="c", subcore_axis_name="s")

    @functools.partial(
        pl.kernel,
        mesh=mesh,
        out_type=jax.ShapeDtypeStruct((B,), jnp.float32),
        scratch_types=[pltpu.VMEM((chunk,), jnp.int32) for _ in range(8)]
        + [pltpu.VMEM((2, 2), jnp.float32), pltpu.VMEM((7, 7), jnp.float32),
           pltpu.VMEM((8, 8), jnp.float32), pltpu.VMEM((17, 17), jnp.float32),
           pltpu.VMEM((chunk,), jnp.float32),
           pltpu.SemaphoreType.DMA, pltpu.SemaphoreType.DMA],
        compiler_params=pltpu.CompilerParams(needs_layout_passes=False),
    )
    def sc(ga, sa, ca, ma, gb, sb, cb, mb, tg, tc_, ts, tm, out,
           ga_v, sa_v, ca_v, ma_v, gb_v, sb_v, cb_v, mb_v,
           tg_v, tc_v, ts_v, tm_v, out_v, sem, sem2):
        wid = lax.axis_index("s") * NC + lax.axis_index("c")
        base = wid * chunk
        half = chunk // 2
        sl1 = pl.ds(base, half)
        sl2 = pl.ds(base + half, half)
        d1 = pl.ds(0, half)
        d2 = pl.ds(half, half)
        idx = [(ga, ga_v), (sa, sa_v), (ca, ca_v), (ma, ma_v),
               (gb, gb_v), (sb, sb_v), (cb, cb_v), (mb, mb_v)]
        copies1 = [pltpu.async_copy(h.at[sl1], v.at[d1], sem)
                   for h, v in idx]
        copies1 += [pltpu.async_copy(tg, tg_v, sem),
                    pltpu.async_copy(tc_, tc_v, sem),
                    pltpu.async_copy(ts, ts_v, sem),
                    pltpu.async_copy(tm, tm_v, sem)]
        copies2 = [pltpu.async_copy(h.at[sl2], v.at[d2], sem2)
                   for h, v in idx]

        def body(r):
            sl = pl.ds(r * L, L)
            v = (plsc.load_gather(tg_v, [ga_v[sl], gb_v[sl]])
                 + plsc.load_gather(tc_v, [ca_v[sl], cb_v[sl]])
                 + plsc.load_gather(ts_v, [sa_v[sl], sb_v[sl]])
                 + plsc.load_gather(tm_v, [ma_v[sl], mb_v[sl]]))
            out_v[sl] = 1.0 / (1.0 + jnp.exp(-v))

        for c in copies1:
            c.wait()
        for r in range(half // L):
            body(r)
        for c in copies2:
            c.wait()
        for r in range(half // L, chunk // L):
            body(r)
        pltpu.sync_copy(out_v, out.at[pl.ds(base, chunk)])

    return sc


def kernel(gA, sA, cA, mA, gB, sB, cB, mB,
           gender_W, college_W, school_W, mbti_W, weight, fc_W, fc_b):
    B = gA.shape[0]
    tg, tc_, ts, tm = _prep_call(gender_W, college_W, school_W, mbti_W,
                                 weight, fc_W, fc_b)
    i32 = jnp.int32
    out = _make_sc_call(B)(
        gA.astype(i32), sA.astype(i32), cA.astype(i32), mA.astype(i32),
        gB.astype(i32), sB.astype(i32), cB.astype(i32), mB.astype(i32),
        tg, tc_, ts, tm)
    return out.reshape(B, 1)


# speedup vs baseline: 1.2949x; 1.0046x over previous
import functools

import jax
import jax.numpy as jnp
from jax import lax
from jax.experimental import pallas as pl
from jax.experimental.pallas import tpu as pltpu
from jax.experimental.pallas import tpu_sc as plsc

_EPS = 1e-8

_OFF_G = 0
_OFF_C = 4
_OFF_S = 53
_OFF_M = 117
_TBL = 416


def _prep_body(gw, cw, sw, mw, w, fcw, fcb, tg, tc_, ts, tm):
    def cosmat(W):
        n = jnp.maximum(jnp.sqrt(jnp.sum(W * W, axis=1, keepdims=True)), _EPS)
        Wn = W / n
        return lax.dot_general(Wn, Wn, (((1,), (1,)), ((), ())),
                               preferred_element_type=jnp.float32)

    a = w[...] * fcw[...]

    tg[...] = cosmat(gw[...]) * a[0:1] + fcb[...]
    tc_[...] = cosmat(cw[...]) * a[1:2]
    ts[...] = cosmat(sw[...]) * a[2:3]
    tm[...] = cosmat(mw[...]) * a[3:4]


def _prep_call(gender_W, college_W, school_W, mbti_W, weight, fc_W, fc_b):
    vmem = pl.BlockSpec(memory_space=pltpu.VMEM)
    return pl.pallas_call(
        _prep_body,
        out_shape=[
            jax.ShapeDtypeStruct((2, 2), jnp.float32),
            jax.ShapeDtypeStruct((7, 7), jnp.float32),
            jax.ShapeDtypeStruct((8, 8), jnp.float32),
            jax.ShapeDtypeStruct((17, 17), jnp.float32),
        ],
        in_specs=[vmem] * 7,
        out_specs=[vmem, vmem, vmem, vmem],
    )(gender_W, college_W, school_W, mbti_W, weight, fc_W.reshape(-1), fc_b)


def _make_sc_call(B):
    info = plsc.get_sparse_core_info()
    NC, NS, L = info.num_cores, info.num_subcores, info.num_lanes
    NW = NC * NS
    chunk = B // NW

    mesh = plsc.VectorSubcoreMesh(core_axis_name="c", subcore_axis_name="s")

    @functools.partial(
        pl.kernel,
        mesh=mesh,
        out_type=jax.ShapeDtypeStruct((B,), jnp.float32),
        scratch_types=[pltpu.VMEM((chunk,), jnp.int32) for _ in range(8)]
        + [pltpu.VMEM((2, 2), jnp.float32), pltpu.VMEM((7, 7), jnp.float32),
           pltpu.VMEM((8, 8), jnp.float32), pltpu.VMEM((17, 17), jnp.float32),
           pltpu.VMEM((chunk,), jnp.float32),
           pltpu.SemaphoreType.DMA, pltpu.SemaphoreType.DMA],
        compiler_params=pltpu.CompilerParams(needs_layout_passes=False),
    )
    def sc(ga, sa, ca, ma, gb, sb, cb, mb, tg, tc_, ts, tm, out,
           ga_v, sa_v, ca_v, ma_v, gb_v, sb_v, cb_v, mb_v,
           tg_v, tc_v, ts_v, tm_v, out_v, sem, sem2):
        wid = lax.axis_index("s") * NC + lax.axis_index("c")
        base = wid * chunk
        half = chunk // 2
        sl1 = pl.ds(base, half)
        sl2 = pl.ds(base + half, half)
        d1 = pl.ds(0, half)
        d2 = pl.ds(half, half)
        idx = [(ga, ga_v), (sa, sa_v), (ca, ca_v), (ma, ma_v),
               (gb, gb_v), (sb, sb_v), (cb, cb_v), (mb, mb_v)]
        copies1 = [pltpu.async_copy(h.at[sl1], v.at[d1], sem)
                   for h, v in idx]
        copies1 += [pltpu.async_copy(tg, tg_v, sem),
                    pltpu.async_copy(tc_, tc_v, sem),
                    pltpu.async_copy(ts, ts_v, sem),
                    pltpu.async_copy(tm, tm_v, sem)]
        copies2 = [pltpu.async_copy(h.at[sl2], v.at[d2], sem2)
                   for h, v in idx]

        def body(r):
            sl = pl.ds(r * L, L)
            v = (plsc.load_gather(tg_v, [ga_v[sl], gb_v[sl]])
                 + plsc.load_gather(tc_v, [ca_v[sl], cb_v[sl]])
                 + plsc.load_gather(ts_v, [sa_v[sl], sb_v[sl]])
                 + plsc.load_gather(tm_v, [ma_v[sl], mb_v[sl]]))
            out_v[sl] = 1.0 / (1.0 + jnp.exp(-v))

        for c in copies1:
            c.wait()
        for r in range(half // L):
            body(r)
        wb1 = pltpu.async_copy(out_v.at[d1], out.at[sl1], sem)
        for c in copies2:
            c.wait()
        for r in range(half // L, chunk // L):
            body(r)
        wb2 = pltpu.async_copy(out_v.at[d2], out.at[sl2], sem2)
        wb1.wait()
        wb2.wait()

    return sc


def kernel(gA, sA, cA, mA, gB, sB, cB, mB,
           gender_W, college_W, school_W, mbti_W, weight, fc_W, fc_b):
    B = gA.shape[0]
    tg, tc_, ts, tm = _prep_call(gender_W, college_W, school_W, mbti_W,
                                 weight, fc_W, fc_b)
    i32 = jnp.int32
    out = _make_sc_call(B)(
        gA.astype(i32), sA.astype(i32), cA.astype(i32), mA.astype(i32),
        gB.astype(i32), sB.astype(i32), cB.astype(i32), mB.astype(i32),
        tg, tc_, ts, tm)
    return out.reshape(B, 1)


# speedup vs baseline: 1.2979x; 1.0023x over previous
import functools

import jax
import jax.numpy as jnp
from jax import lax
from jax.experimental import pallas as pl
from jax.experimental.pallas import tpu as pltpu
from jax.experimental.pallas import tpu_sc as plsc

_EPS = 1e-8


def _prep_body(gw, cw, sw, mw, w, fcw, fcb, tg, tc_, ts, tm):
    def cosmat(W):
        n = jnp.maximum(jnp.sqrt(jnp.sum(W * W, axis=1, keepdims=True)), _EPS)
        Wn = W / n
        return lax.dot_general(Wn, Wn, (((1,), (1,)), ((), ())),
                               preferred_element_type=jnp.float32)

    a = w[...] * fcw[...]

    tg[...] = cosmat(gw[...]) * a[0:1] + fcb[...]
    tc_[...] = cosmat(cw[...]) * a[1:2]
    ts[...] = cosmat(sw[...]) * a[2:3]
    tm[...] = cosmat(mw[...]) * a[3:4]


def _prep_call(gender_W, college_W, school_W, mbti_W, weight, fc_W, fc_b):
    vmem = pl.BlockSpec(memory_space=pltpu.VMEM)
    return pl.pallas_call(
        _prep_body,
        out_shape=[
            jax.ShapeDtypeStruct((2, 2), jnp.float32),
            jax.ShapeDtypeStruct((7, 7), jnp.float32),
            jax.ShapeDtypeStruct((8, 8), jnp.float32),
            jax.ShapeDtypeStruct((17, 17), jnp.float32),
        ],
        in_specs=[vmem] * 7,
        out_specs=[vmem, vmem, vmem, vmem],
    )(gender_W, college_W, school_W, mbti_W, weight, fc_W.reshape(-1), fc_b)


def _make_sc_call(B):
    info = plsc.get_sparse_core_info()
    NC, NS, L = info.num_cores, info.num_subcores, info.num_lanes
    NW = NC * NS
    chunk = B // NW

    mesh = plsc.VectorSubcoreMesh(core_axis_name="c", subcore_axis_name="s")

    @functools.partial(
        pl.kernel,
        mesh=mesh,
        out_type=jax.ShapeDtypeStruct((B,), jnp.float32),
        scratch_types=[pltpu.VMEM((chunk,), jnp.int32) for _ in range(8)]
        + [pltpu.VMEM((2, 2), jnp.float32), pltpu.VMEM((7, 7), jnp.float32),
           pltpu.VMEM((8, 8), jnp.float32), pltpu.VMEM((17, 17), jnp.float32),
           pltpu.VMEM((chunk,), jnp.float32),
           pltpu.SemaphoreType.DMA, pltpu.SemaphoreType.DMA],
        compiler_params=pltpu.CompilerParams(needs_layout_passes=False),
    )
    def sc(ga, sa, ca, ma, gb, sb, cb, mb, tg, tc_, ts, tm, out,
           ga_v, sa_v, ca_v, ma_v, gb_v, sb_v, cb_v, mb_v,
           tg_v, tc_v, ts_v, tm_v, out_v, sem, sem2):
        wid = lax.axis_index("s") * NC + lax.axis_index("c")
        base = wid * chunk
        half = chunk // 2
        sl1 = pl.ds(base, half)
        sl2 = pl.ds(base + half, half)
        d1 = pl.ds(0, half)
        d2 = pl.ds(half, half)
        idx = [(ga, ga_v), (sa, sa_v), (ca, ca_v), (ma, ma_v),
               (gb, gb_v), (sb, sb_v), (cb, cb_v), (mb, mb_v)]
        copies1 = [pltpu.async_copy(h.at[sl1], v.at[d1], sem)
                   for h, v in idx]
        copies1 += [pltpu.async_copy(tg, tg_v, sem),
                    pltpu.async_copy(tc_, tc_v, sem),
                    pltpu.async_copy(ts, ts_v, sem),
                    pltpu.async_copy(tm, tm_v, sem)]
        copies2 = [pltpu.async_copy(h.at[sl2], v.at[d2], sem2)
                   for h, v in idx]

        def body(r):
            sl = pl.ds(r * L, L)
            v = (plsc.load_gather(tg_v, [ga_v[sl], gb_v[sl]])
                 + plsc.load_gather(tc_v, [ca_v[sl], cb_v[sl]])
                 + plsc.load_gather(ts_v, [sa_v[sl], sb_v[sl]])
                 + plsc.load_gather(tm_v, [ma_v[sl], mb_v[sl]]))
            out_v[sl] = 1.0 / (1.0 + jnp.exp(-v))

        for c in copies1:
            c.wait()
        for r in range(half // L):
            body(r)
        wb1 = pltpu.async_copy(out_v.at[d1], out.at[sl1], sem)
        for c in copies2:
            c.wait()
        for r in range(half // L, chunk // L):
            body(r)
        wb2 = pltpu.async_copy(out_v.at[d2], out.at[sl2], sem2)
        wb1.wait()
        wb2.wait()

    return sc


def kernel(gA, sA, cA, mA, gB, sB, cB, mB,
           gender_W, college_W, school_W, mbti_W, weight, fc_W, fc_b):
    B = gA.shape[0]
    tg, tc_, ts, tm = _prep_call(gender_W, college_W, school_W, mbti_W,
                                 weight, fc_W, fc_b)
    i32 = jnp.int32
    out = _make_sc_call(B)(
        gA.astype(i32), sA.astype(i32), cA.astype(i32), mA.astype(i32),
        gB.astype(i32), sB.astype(i32), cB.astype(i32), mB.astype(i32),
        tg, tc_, ts, tm)
    return out.reshape(B, 1)
